# baseline (speedup 1.0000x reference)
import jax
import jax.numpy as jnp
from jax.experimental import pallas as pl

_P_PAD = 1024
_PAD_VAL = 1e18


def _block_kernel(rel_ref, x_ref, p_ref, pout_ref, out_ref):
    bB = x_ref.shape[0]
    xb = x_ref[...]
    p = p_ref[...]
    w = rel_ref[...] * rel_ref[...]

    dn = (((1,), (0,)), ((), ()))
    s = jax.lax.dot_general(xb * (-2.0 * w), p, dn,
                            precision=jax.lax.Precision.HIGHEST,
                            preferred_element_type=jnp.float32)
    pnorm = jax.lax.dot_general(w, p * p, dn,
                                precision=jax.lax.Precision.HIGHEST,
                                preferred_element_type=jnp.float32)
    dist = pnorm + s

    m = jnp.min(dist, axis=1, keepdims=True)
    iota = jax.lax.broadcasted_iota(jnp.int32, (bB, _P_PAD), 1)
    cand = jnp.where(dist == m, iota, jnp.int32(2**30))
    j = jnp.min(cand, axis=1, keepdims=True)
    pout = pout_ref[...]
    sel = jnp.where(iota == j, pout, 0.0)
    out_ref[...] = jnp.sum(sel, axis=1, keepdims=True)


def kernel(x, prototypes, prototype_outputs, relevance):
    B, D = x.shape
    P = prototypes.shape[0]
    bB = 4096

    pt = jnp.pad(prototypes, ((0, _P_PAD - P), (0, 0)),
                 constant_values=_PAD_VAL).T
    pout = jnp.pad(prototype_outputs, ((0, _P_PAD - P), (0, 0))).reshape(1, _P_PAD)

    out = pl.pallas_call(
        _block_kernel,
        grid=(B // bB,),
        in_specs=[
            pl.BlockSpec((1, D), lambda i: (0, 0)),
            pl.BlockSpec((bB, D), lambda i: (i, 0)),
            pl.BlockSpec((D, _P_PAD), lambda i: (0, 0)),
            pl.BlockSpec((1, _P_PAD), lambda i: (0, 0)),
        ],
        out_specs=pl.BlockSpec((bB, 1), lambda i: (i, 0)),
        out_shape=jax.ShapeDtypeStruct((B, 1), jnp.float32),
    )(relevance.reshape(1, D), x, pt, pout)
    return out


# speedup vs baseline: 1.0525x; 1.0525x over previous
import jax
import jax.numpy as jnp
from jax.experimental import pallas as pl
from jax.experimental.pallas import tpu as pltpu

_P_PAD = 1024
_PAD_VAL = 1e18


def _block_kernel(rel_ref, x_ref, praw_ref, pout_ref, out_ref,
                  pt_scr, pn_scr, po_scr):
    bB = x_ref.shape[0]
    P, D = praw_ref.shape
    dn = (((1,), (0,)), ((), ()))
    hi = jax.lax.Precision.HIGHEST
    w = rel_ref[...] * rel_ref[...]

    @pl.when(pl.program_id(0) == 0)
    def _stage():
        pT = praw_ref[...].T
        pad = jnp.full((D, _P_PAD - P), _PAD_VAL, jnp.float32)
        pt = jnp.concatenate([pT, pad], axis=1)
        pt_scr[...] = pt
        pn_scr[...] = jax.lax.dot_general(
            w, pt * pt, dn, precision=hi,
            preferred_element_type=jnp.float32)
        poT = pout_ref[...].T
        po_scr[...] = jnp.concatenate(
            [poT, jnp.zeros((1, _P_PAD - P), jnp.float32)], axis=1)

    xb = x_ref[...]
    s = jax.lax.dot_general(xb * (-2.0 * w), pt_scr[...], dn, precision=hi,
                            preferred_element_type=jnp.float32)
    dist = pn_scr[...] + s

    m = jnp.min(dist, axis=1, keepdims=True)
    iota = jax.lax.broadcasted_iota(jnp.int32, (bB, _P_PAD), 1)
    cand = jnp.where(dist == m, iota, jnp.int32(2**30))
    j = jnp.min(cand, axis=1, keepdims=True)
    sel = jnp.where(iota == j, po_scr[...], 0.0)
    out_ref[...] = jnp.sum(sel, axis=1, keepdims=True)


def kernel(x, prototypes, prototype_outputs, relevance):
    B, D = x.shape
    bB = 1024

    out = pl.pallas_call(
        _block_kernel,
        grid=(B // bB,),
        in_specs=[
            pl.BlockSpec((1, D), lambda i: (0, 0)),
            pl.BlockSpec((bB, D), lambda i: (i, 0)),
            pl.BlockSpec(prototypes.shape, lambda i: (0, 0)),
            pl.BlockSpec(prototype_outputs.shape, lambda i: (0, 0)),
        ],
        out_specs=pl.BlockSpec((bB, 1), lambda i: (i, 0)),
        out_shape=jax.ShapeDtypeStruct((B, 1), jnp.float32),
        scratch_shapes=[
            pltpu.VMEM((D, _P_PAD), jnp.float32),
            pltpu.VMEM((1, _P_PAD), jnp.float32),
            pltpu.VMEM((1, _P_PAD), jnp.float32),
        ],
    )(relevance.reshape(1, D), x, prototypes, prototype_outputs)
    return out
